# baseline (speedup 1.0000x reference)
import functools

import jax
import jax.numpy as jnp
from jax import lax
from jax.experimental import pallas as pl
from jax.experimental.pallas import tpu as pltpu

_B = 1048576
_R = _B // 128
_BLK = 1024
_CHUNK = _BLK * 128
_STEPS = _R // _BLK

_EPS = 1e-12


def _loss_kernel(lab_ref, plab_ref, lab1_ref, go_ref, po_ref,
                 glt_hbm, plt_hbm, out_ref,
                 g8_buf, p8_buf, g2_buf, p2_buf, sems, acc_ref):
    i = pl.program_id(0)

    def _copies(step, slot):
        base = step * _CHUNK
        return (
            pltpu.make_async_copy(
                glt_hbm.at[pl.ds(0, 8), pl.ds(base, _CHUNK)],
                g8_buf.at[slot], sems.at[slot, 0]),
            pltpu.make_async_copy(
                plt_hbm.at[pl.ds(0, 8), pl.ds(base, _CHUNK)],
                p8_buf.at[slot], sems.at[slot, 1]),
            pltpu.make_async_copy(
                glt_hbm.at[pl.ds(8, 2), pl.ds(base, _CHUNK)],
                g2_buf.at[slot], sems.at[slot, 2]),
            pltpu.make_async_copy(
                plt_hbm.at[pl.ds(8, 2), pl.ds(base, _CHUNK)],
                p2_buf.at[slot], sems.at[slot, 3]),
        )

    slot = lax.rem(i, 2)

    @pl.when(i == 0)
    def _prime():
        for c in _copies(0, 0):
            c.start()

    @pl.when(i < _STEPS - 1)
    def _prefetch():
        for c in _copies(i + 1, lax.rem(i + 1, 2)):
            c.start()

    for c in _copies(i, slot):
        c.wait()

    label = lab_ref[...] - 2
    t = label.astype(jnp.float32)
    mask_cls = (label >= 0).astype(jnp.float32)
    mask_box = (label != 0).astype(jnp.float32)

    p = jnp.clip(plab_ref[...], _EPS, 1.0 - _EPS)
    bce = -(t * jnp.log(p) + (1.0 - t) * jnp.log(1.0 - p))
    s_bce = jnp.sum(mask_cls * bce)
    n_cls = jnp.sum(mask_cls)
    n_box = jnp.sum(mask_box)

    d = po_ref[...] - go_ref[...]
    rs_box = jnp.sum(d * d, axis=1)
    s_box = jnp.sum(mask_box * rs_box)

    lab1 = lab1_ref[...]
    mask_lmk_t = (lab1 == 0).astype(jnp.float32)
    n_lmk = jnp.sum(mask_lmk_t)
    d8 = p8_buf[slot] - g8_buf[slot]
    d2 = p2_buf[slot] - g2_buf[slot]
    rs_lmk = jnp.sum(d8 * d8, axis=0) + jnp.sum(d2 * d2, axis=0)
    s_lmk = jnp.sum(mask_lmk_t * rs_lmk)

    @pl.when(i == 0)
    def _init():
        for k in range(6):
            acc_ref[k] = 0.0

    acc_ref[0] += s_bce
    acc_ref[1] += n_cls
    acc_ref[2] += s_box
    acc_ref[3] += n_box
    acc_ref[4] += s_lmk
    acc_ref[5] += n_lmk

    @pl.when(i == _STEPS - 1)
    def _fin():
        cls_loss = acc_ref[0] / acc_ref[1]
        box_loss = acc_ref[2] / (acc_ref[3] * 4.0)
        lmk_loss = acc_ref[4] / (acc_ref[5] * 10.0)
        total = cls_loss + box_loss + lmk_loss
        out_ref[...] = jnp.full((1, 1), total, dtype=jnp.float32)


def _native_view(x, c):
    return x.reshape(_R, 128, c).transpose(0, 2, 1)


@functools.partial(jax.jit)
def kernel(gt_label, pred_label, gt_offset, pred_offset, gt_landmark,
           pred_landmark):
    lab32 = gt_label.astype(jnp.int32)
    lab = lab32.reshape(_R, 128)
    plab = pred_label.reshape(_R, 128)
    go = _native_view(gt_offset, 4)
    po = _native_view(pred_offset, 4)
    glt = gt_landmark.T
    plt = pred_landmark.T

    out = pl.pallas_call(
        _loss_kernel,
        grid=(_STEPS,),
        in_specs=[
            pl.BlockSpec((_BLK, 128), lambda i: (i, 0)),
            pl.BlockSpec((_BLK, 128), lambda i: (i, 0)),
            pl.BlockSpec((_CHUNK,), lambda i: (i,)),
            pl.BlockSpec((_BLK, 4, 128), lambda i: (i, 0, 0)),
            pl.BlockSpec((_BLK, 4, 128), lambda i: (i, 0, 0)),
            pl.BlockSpec(memory_space=pl.ANY),
            pl.BlockSpec(memory_space=pl.ANY),
        ],
        out_specs=pl.BlockSpec((1, 1), lambda i: (0, 0)),
        out_shape=jax.ShapeDtypeStruct((1, 1), jnp.float32),
        scratch_shapes=[
            pltpu.VMEM((2, 8, _CHUNK), jnp.float32),
            pltpu.VMEM((2, 8, _CHUNK), jnp.float32),
            pltpu.VMEM((2, 2, _CHUNK), jnp.float32),
            pltpu.VMEM((2, 2, _CHUNK), jnp.float32),
            pltpu.SemaphoreType.DMA((2, 4)),
            pltpu.SMEM((8,), jnp.float32),
        ],
    )(lab, plab, lab32, go, po, glt, plt)
    return out.reshape(())


# speedup vs baseline: 1.0353x; 1.0353x over previous
import functools

import jax
import jax.numpy as jnp
from jax import lax
from jax.experimental import pallas as pl
from jax.experimental.pallas import tpu as pltpu

_B = 1048576
_R = _B // 128
_BLK = 512
_CHUNK = _BLK * 128
_STEPS = _R // _BLK

_EPS = 1e-12


def _loss_kernel(lab_ref, plab_ref, lab1_ref, go_ref, po_ref,
                 glt_hbm, plt_hbm, out_ref,
                 g8_buf, p8_buf, g2_buf, p2_buf, sems, acc_ref):
    i = pl.program_id(0)

    def _copies(step, slot):
        base = step * _CHUNK
        return (
            pltpu.make_async_copy(
                glt_hbm.at[pl.ds(0, 8), pl.ds(base, _CHUNK)],
                g8_buf.at[slot], sems.at[slot, 0]),
            pltpu.make_async_copy(
                plt_hbm.at[pl.ds(0, 8), pl.ds(base, _CHUNK)],
                p8_buf.at[slot], sems.at[slot, 1]),
            pltpu.make_async_copy(
                glt_hbm.at[pl.ds(8, 2), pl.ds(base, _CHUNK)],
                g2_buf.at[slot], sems.at[slot, 2]),
            pltpu.make_async_copy(
                plt_hbm.at[pl.ds(8, 2), pl.ds(base, _CHUNK)],
                p2_buf.at[slot], sems.at[slot, 3]),
        )

    slot = lax.rem(i, 2)

    @pl.when(i == 0)
    def _prime():
        for c in _copies(0, 0):
            c.start()

    @pl.when(i < _STEPS - 1)
    def _prefetch():
        for c in _copies(i + 1, lax.rem(i + 1, 2)):
            c.start()

    for c in _copies(i, slot):
        c.wait()

    label = lab_ref[...] - 2
    t = label.astype(jnp.float32)
    mask_cls = (label >= 0).astype(jnp.float32)
    mask_box = (label != 0).astype(jnp.float32)

    p = jnp.clip(plab_ref[...], _EPS, 1.0 - _EPS)
    bce = -(t * jnp.log(p) + (1.0 - t) * jnp.log(1.0 - p))
    s_bce = jnp.sum(mask_cls * bce)
    n_cls = jnp.sum(mask_cls)
    n_box = jnp.sum(mask_box)

    d = po_ref[...] - go_ref[...]
    rs_box = jnp.sum(d * d, axis=1)
    s_box = jnp.sum(mask_box * rs_box)

    lab1 = lab1_ref[...]
    mask_lmk_t = (lab1 == 0).astype(jnp.float32)
    n_lmk = jnp.sum(mask_lmk_t)
    d8 = p8_buf[slot] - g8_buf[slot]
    d2 = p2_buf[slot] - g2_buf[slot]
    rs_lmk = jnp.sum(d8 * d8, axis=0) + jnp.sum(d2 * d2, axis=0)
    s_lmk = jnp.sum(mask_lmk_t * rs_lmk)

    @pl.when(i == 0)
    def _init():
        for k in range(6):
            acc_ref[k] = 0.0

    acc_ref[0] += s_bce
    acc_ref[1] += n_cls
    acc_ref[2] += s_box
    acc_ref[3] += n_box
    acc_ref[4] += s_lmk
    acc_ref[5] += n_lmk

    @pl.when(i == _STEPS - 1)
    def _fin():
        cls_loss = acc_ref[0] / acc_ref[1]
        box_loss = acc_ref[2] / (acc_ref[3] * 4.0)
        lmk_loss = acc_ref[4] / (acc_ref[5] * 10.0)
        total = cls_loss + box_loss + lmk_loss
        out_ref[...] = jnp.full((1, 1), total, dtype=jnp.float32)


def _native_view(x, c):
    return x.reshape(_R, 128, c).transpose(0, 2, 1)


@functools.partial(jax.jit)
def kernel(gt_label, pred_label, gt_offset, pred_offset, gt_landmark,
           pred_landmark):
    lab32 = gt_label.astype(jnp.int32)
    lab = lab32.reshape(_R, 128)
    plab = pred_label.reshape(_R, 128)
    go = _native_view(gt_offset, 4)
    po = _native_view(pred_offset, 4)
    glt = gt_landmark.T
    plt = pred_landmark.T

    out = pl.pallas_call(
        _loss_kernel,
        grid=(_STEPS,),
        in_specs=[
            pl.BlockSpec((_BLK, 128), lambda i: (i, 0)),
            pl.BlockSpec((_BLK, 128), lambda i: (i, 0)),
            pl.BlockSpec((_CHUNK,), lambda i: (i,)),
            pl.BlockSpec((_BLK, 4, 128), lambda i: (i, 0, 0)),
            pl.BlockSpec((_BLK, 4, 128), lambda i: (i, 0, 0)),
            pl.BlockSpec(memory_space=pl.ANY),
            pl.BlockSpec(memory_space=pl.ANY),
        ],
        out_specs=pl.BlockSpec((1, 1), lambda i: (0, 0)),
        out_shape=jax.ShapeDtypeStruct((1, 1), jnp.float32),
        scratch_shapes=[
            pltpu.VMEM((2, 8, _CHUNK), jnp.float32),
            pltpu.VMEM((2, 8, _CHUNK), jnp.float32),
            pltpu.VMEM((2, 2, _CHUNK), jnp.float32),
            pltpu.VMEM((2, 2, _CHUNK), jnp.float32),
            pltpu.SemaphoreType.DMA((2, 4)),
            pltpu.SMEM((8,), jnp.float32),
        ],
    )(lab, plab, lab32, go, po, glt, plt)
    return out.reshape(())
